# baseline (speedup 1.0000x reference)
import functools

import jax
import jax.numpy as jnp
from jax import lax
from jax.experimental import pallas as pl
from jax.experimental.pallas import tpu as pltpu
from jax.experimental.pallas import tpu_sc as plsc

N = 50000
VIDEO_IN = 512
TAG_IN = 64
HIDDEN = 128

_NC = 2
_NS = 16
_NW = _NC * _NS
_CHUNK = 128
_KPW = 13
_BPW = _CHUNK * _KPW
_BPAD = _NW * _BPW
_RING = 16
_SLAB = 8


def _sc_gather_native(table, idx_pad):
    mesh = plsc.VectorSubcoreMesh(core_axis_name="c", subcore_axis_name="s")

    @functools.partial(
        pl.kernel,
        out_type=jax.ShapeDtypeStruct((_BPAD, TAG_IN), jnp.float32),
        mesh=mesh,
        scratch_types=[
            pltpu.VMEM((_KPW, _CHUNK), jnp.int32),
            pltpu.VMEM((_RING, _SLAB, TAG_IN), jnp.float32),
            pltpu.VMEM((_CHUNK, TAG_IN), jnp.float32),
            pltpu.SemaphoreType.DMA,
        ],
    )
    def k(table_hbm, idx_hbm, out_hbm, idx_v, slab_v, rows_v, sem):
        wid = lax.axis_index("s") * _NC + lax.axis_index("c")
        base_out = wid * _BPW
        pltpu.sync_copy(idx_hbm.at[wid], idx_v)

        n_groups = _BPW // 16

        def idx_vec(g):
            gc = jnp.minimum(g, n_groups - 1)
            return idx_v[gc >> 3, pl.ds((gc & 7) * 16, 16)]

        def slab_start(i, k):
            row0 = pl.multiple_of((i >> 3) * _SLAB, _SLAB)
            pltpu.make_async_copy(
                table_hbm.at[pl.ds(row0, _SLAB)],
                slab_v.at[k],
                sem,
            ).start()

        vec0 = idx_vec(0)
        for k in range(16):
            slab_start(vec0[k], k)

        def body(g, carry):
            cur = idx_vec(g)
            nxt = idx_vec(g + 1)
            not_last = g < n_groups - 1
            for k in range(16):
                pltpu.make_async_copy(
                    table_hbm.at[pl.ds(0, _SLAB)],
                    slab_v.at[k],
                    sem,
                ).wait()
                r_in = cur[k] & (_SLAB - 1)
                row = (g & 7) * 16 + k
                for c in range(TAG_IN // 16):
                    rows_v[row, pl.ds(c * 16, 16)] = slab_v[k, r_in,
                                                            pl.ds(c * 16, 16)]

                @pl.when(not_last)
                def _():
                    slab_start(nxt[k], k)

            @pl.when((g & 7) == 7)
            def _():
                off = pl.multiple_of(base_out + (g >> 3) * _CHUNK, 8)
                pltpu.sync_copy(rows_v, out_hbm.at[pl.ds(off, _CHUNK)])

            return carry

        lax.fori_loop(0, n_groups, body, 0)

    return k(table, idx_pad)


_BLK = 2000
_GRID = N // _BLK


def _video_stats_body(v_ref, vs_ref, vq_ref):
    i = pl.program_id(0)

    @pl.when(i == 0)
    def _():
        vs_ref[...] = jnp.zeros_like(vs_ref)
        vq_ref[...] = jnp.zeros_like(vq_ref)

    x = v_ref[...]
    vs_ref[...] += jnp.sum(x, axis=0, keepdims=True)
    vq_ref[...] += jnp.sum(x * x, axis=0, keepdims=True)


def _tag_stats_body(t_ref, g_ref, ts_ref, tq_ref):
    i = pl.program_id(0)

    @pl.when(i == 0)
    def _():
        ts_ref[...] = jnp.zeros_like(ts_ref)
        tq_ref[...] = jnp.zeros_like(tq_ref)

    t = t_ref[...] + g_ref[...]
    ts_ref[...] += jnp.sum(t, axis=0, keepdims=True)
    tq_ref[...] += jnp.sum(t * t, axis=0, keepdims=True)


def _gelu(x):
    return 0.5 * x * (1.0 + lax.erf(x * 0.7071067811865476))


def _branch(x, sum_ref, sq_ref, g_ref, b_ref, w_ref, bias_ref):
    n = jnp.float32(N)
    mu = sum_ref[...] / n
    var = sq_ref[...] / n - mu * mu
    s = g_ref[...] / jnp.sqrt(var + 1e-5)
    shift = b_ref[...] - mu * s
    w = w_ref[...]
    acc = jnp.dot(x * s, w, preferred_element_type=jnp.float32)
    bias = jnp.dot(shift, w, preferred_element_type=jnp.float32) + bias_ref[...]
    return _gelu(acc + bias)


def _video_apply_body(v_ref, vs_ref, vq_ref, vg_ref, vb_ref, wv_ref, bv_ref,
                      hv_ref):
    hv_ref[...] = _branch(v_ref[...], vs_ref, vq_ref, vg_ref, vb_ref,
                          wv_ref, bv_ref)


def _tag_apply_body(t_ref, g_ref, ts_ref, tq_ref,
                    tg_ref, tb_ref, wt_ref, bt_ref, ht_ref):
    t = t_ref[...] + g_ref[...]
    ht_ref[...] = _branch(t, ts_ref, tq_ref, tg_ref, tb_ref, wt_ref, bt_ref)


def _row_spec(rows, cols):
    return pl.BlockSpec((rows, cols), lambda i: (i, 0))


def _const_spec(rows, cols):
    return pl.BlockSpec((rows, cols), lambda i: (0, 0))


_ARB = pltpu.CompilerParams(dimension_semantics=("arbitrary",))


def kernel(video_feat, tag_feat, tag_nids, tag_table,
           v_bn_g, v_bn_b, W_v, b_v, t_bn_g, t_bn_b, W_t, b_t):
    nids = tag_nids.astype(jnp.int32)
    idx_pad = jnp.concatenate(
        [nids, jnp.zeros((_BPAD - N,), jnp.int32)]
    ).reshape(_NW, _KPW, _CHUNK)

    gathered = _sc_gather_native(tag_table, idx_pad)

    vs, vq = pl.pallas_call(
        _video_stats_body,
        grid=(_GRID,),
        in_specs=[_row_spec(_BLK, VIDEO_IN)],
        out_specs=[_const_spec(1, VIDEO_IN), _const_spec(1, VIDEO_IN)],
        out_shape=[
            jax.ShapeDtypeStruct((1, VIDEO_IN), jnp.float32),
            jax.ShapeDtypeStruct((1, VIDEO_IN), jnp.float32),
        ],
        compiler_params=_ARB,
    )(video_feat)

    hv = pl.pallas_call(
        _video_apply_body,
        grid=(_GRID,),
        in_specs=[
            _row_spec(_BLK, VIDEO_IN),
            _const_spec(1, VIDEO_IN), _const_spec(1, VIDEO_IN),
            _const_spec(1, VIDEO_IN), _const_spec(1, VIDEO_IN),
            pl.BlockSpec((VIDEO_IN, HIDDEN), lambda i: (0, 0)),
            _const_spec(1, HIDDEN),
        ],
        out_specs=_row_spec(_BLK, HIDDEN),
        out_shape=jax.ShapeDtypeStruct((N, HIDDEN), jnp.float32),
        compiler_params=_ARB,
    )(video_feat, vs, vq,
      v_bn_g.reshape(1, VIDEO_IN), v_bn_b.reshape(1, VIDEO_IN), W_v,
      b_v.reshape(1, HIDDEN))

    ts, tq = pl.pallas_call(
        _tag_stats_body,
        grid=(_GRID,),
        in_specs=[
            _row_spec(_BLK, TAG_IN),
            _row_spec(_BLK, TAG_IN),
        ],
        out_specs=[_const_spec(1, TAG_IN), _const_spec(1, TAG_IN)],
        out_shape=[
            jax.ShapeDtypeStruct((1, TAG_IN), jnp.float32),
            jax.ShapeDtypeStruct((1, TAG_IN), jnp.float32),
        ],
        compiler_params=_ARB,
    )(tag_feat, gathered)

    ht = pl.pallas_call(
        _tag_apply_body,
        grid=(_GRID,),
        in_specs=[
            _row_spec(_BLK, TAG_IN),
            _row_spec(_BLK, TAG_IN),
            _const_spec(1, TAG_IN), _const_spec(1, TAG_IN),
            _const_spec(1, TAG_IN), _const_spec(1, TAG_IN),
            pl.BlockSpec((TAG_IN, HIDDEN), lambda i: (0, 0)),
            _const_spec(1, HIDDEN),
        ],
        out_specs=_row_spec(_BLK, HIDDEN),
        out_shape=jax.ShapeDtypeStruct((N, HIDDEN), jnp.float32),
        compiler_params=_ARB,
    )(tag_feat, gathered,
      ts, tq,
      t_bn_g.reshape(1, TAG_IN), t_bn_b.reshape(1, TAG_IN), W_t,
      b_t.reshape(1, HIDDEN))

    return (hv, ht)
